# baseline (speedup 1.0000x reference)
import functools

import jax
import jax.numpy as jnp
from jax import lax
from jax.experimental import pallas as pl
from jax.experimental.pallas import tpu as pltpu
from jax.experimental.pallas import tpu_sc as plsc

EPS = 1e-5
NC = 2
NS = 16
LANES = 16
CHUNK = 128
ZROWS = 128


def _sc_scatter_body(npad, nchunk, nhalf, affine, nf,
                     x2, srcr, dstr, ewr, ss, h3,
                     acc, src_v, dst_v, ew_v, ss_v, rows0, rows1,
                     g0, g1, s0, s1):
    c = lax.axis_index("c")
    s = lax.axis_index("s")
    rows_per_tile = npad // NS
    nzcopy = rows_per_tile // ZROWS
    hchunk = nchunk // nhalf
    nvec = nf // LANES

    pltpu.sync_copy(ss, ss_v)

    zero16 = jnp.zeros((LANES,), jnp.float32)

    def zrow(i, carry):
        for k in range(nvec):
            rows0[i, pl.ds(k * LANES, LANES)] = zero16
        return carry

    lax.fori_loop(0, ZROWS, zrow, 0)
    base = s * rows_per_tile
    for z in range(nzcopy):
        pltpu.sync_copy(rows0, acc.at[pl.ds(base + z * ZROWS, ZROWS)])
    plsc.subcore_barrier()

    def scale_rows(buf, k):
        def group(g, gcarry):
            wv = ew_v[k, pl.ds(g * LANES, LANES)]
            for l in range(LANES):
                w = wv[l]
                i = g * LANES + l
                for q in range(nvec):
                    sl = pl.ds(q * LANES, LANES)
                    v = buf[i, sl]
                    if affine:
                        v = jnp.maximum(v * ss_v[0, sl] + ss_v[1, sl], 0.0)
                    buf[i, sl] = v * w
            return gcarry

        lax.fori_loop(0, CHUNK // LANES, group, 0)

    for half in range(nhalf):
        hbase = half * hchunk
        pltpu.sync_copy(srcr.at[c, s, pl.ds(hbase, hchunk)], src_v)
        pltpu.sync_copy(dstr.at[s, pl.ds(hbase, hchunk)], dst_v)
        pltpu.sync_copy(ewr.at[c, s, pl.ds(hbase, hchunk)], ew_v)

        pltpu.async_copy(x2.at[src_v.at[0]], rows0, g0)

        def pipe(k2, carry):
            k = 2 * k2
            @pl.when(k2 >= 1)
            def _():
                pltpu.make_async_copy(rows1, acc.at[dst_v.at[k - 1]],
                                      s1).wait()
            pltpu.async_copy(x2.at[src_v.at[k + 1]], rows1, g1)
            pltpu.make_async_copy(x2.at[src_v.at[k]], rows0, g0).wait()
            scale_rows(rows0, k)
            pltpu.async_copy(rows0, acc.at[dst_v.at[k]], s0, add=True)
            @pl.when(k2 < hchunk // 2 - 1)
            def _():
                pltpu.make_async_copy(rows0, acc.at[dst_v.at[k]], s0).wait()
                pltpu.async_copy(x2.at[src_v.at[k + 2]], rows0, g0)
            pltpu.make_async_copy(x2.at[src_v.at[k + 1]], rows1, g1).wait()
            scale_rows(rows1, k + 1)
            pltpu.async_copy(rows1, acc.at[dst_v.at[k + 1]], s1, add=True)
            return carry

        lax.fori_loop(0, hchunk // 2, pipe, 0)
        pltpu.make_async_copy(rows0, acc.at[dst_v.at[hchunk - 2]], s0).wait()
        pltpu.make_async_copy(rows1, acc.at[dst_v.at[hchunk - 1]], s1).wait()

    plsc.subcore_barrier()

    out_base = c * npad + base
    for z in range(nzcopy):
        pltpu.sync_copy(acc.at[pl.ds(base + z * ZROWS, ZROWS)],
                        h3.at[pl.ds(out_base + z * ZROWS, ZROWS)])


def _sc_scatter(x2, srcr, dstr, ewr, ss, npad, affine):
    f = x2.shape[1]
    nchunk = srcr.shape[2]
    nhalf = 2
    hchunk = nchunk // nhalf
    mesh = plsc.VectorSubcoreMesh(core_axis_name="c", subcore_axis_name="s",
                                  num_cores=NC, num_subcores=NS)
    return pl.kernel(
        functools.partial(_sc_scatter_body, npad, nchunk, nhalf, affine, f),
        out_type=jax.ShapeDtypeStruct((NC * npad, f), jnp.float32),
        mesh=mesh,
        scratch_types=[
            pltpu.VMEM_SHARED((npad, f), jnp.float32),
            pltpu.VMEM((hchunk, CHUNK), jnp.int32),
            pltpu.VMEM((hchunk, CHUNK), jnp.int32),
            pltpu.VMEM((hchunk, CHUNK), jnp.float32),
            pltpu.VMEM((2, f), jnp.float32),
            pltpu.VMEM((CHUNK, f), jnp.float32),
            pltpu.VMEM((CHUNK, f), jnp.float32),
            pltpu.SemaphoreType.DMA,
            pltpu.SemaphoreType.DMA,
            pltpu.SemaphoreType.DMA,
            pltpu.SemaphoreType.DMA,
        ],
    )(x2, srcr, dstr, ewr, ss)


def _mm_stats_body(cnt, x_ref, w_ref, b_ref, g_ref, be_ref,
                   y_ref, ss_ref, acc_ref):
    t = pl.program_id(0)
    i = pl.program_id(1)
    y = jnp.dot(x_ref[0], w_ref[...],
                preferred_element_type=jnp.float32) + b_ref[...]
    y_ref[...] = y

    @pl.when((t == 0) & (i == 0))
    def _():
        acc_ref[...] = jnp.zeros_like(acc_ref)

    acc_ref[0:1, :] += jnp.sum(y, axis=0, keepdims=True)
    acc_ref[1:2, :] += jnp.sum(y * y, axis=0, keepdims=True)

    @pl.when((t == pl.num_programs(0) - 1) & (i == pl.num_programs(1) - 1))
    def _():
        mean = acc_ref[0:1, :] / cnt
        var = acc_ref[1:2, :] / cnt - mean * mean
        scale = g_ref[...] * lax.rsqrt(var + EPS)
        ss_ref[0:1, :] = scale
        ss_ref[1:2, :] = be_ref[...] - mean * scale


def _tc_mm_stats(h3, w, b, g, be, nvalid, br):
    t_, npad, f = h3.shape
    grid = (t_, nvalid // br)
    return pl.pallas_call(
        functools.partial(_mm_stats_body, float(t_ * nvalid)),
        grid=grid,
        in_specs=[
            pl.BlockSpec((1, br, f), lambda t, i: (t, i, 0)),
            pl.BlockSpec((f, f), lambda t, i: (0, 0)),
            pl.BlockSpec((1, f), lambda t, i: (0, 0)),
            pl.BlockSpec((1, f), lambda t, i: (0, 0)),
            pl.BlockSpec((1, f), lambda t, i: (0, 0)),
        ],
        out_specs=[
            pl.BlockSpec((br, f), lambda t, i: (t * (nvalid // br) + i, 0)),
            pl.BlockSpec((2, f), lambda t, i: (0, 0)),
        ],
        out_shape=[
            jax.ShapeDtypeStruct((t_ * nvalid, f), jnp.float32),
            jax.ShapeDtypeStruct((2, f), jnp.float32),
        ],
        scratch_shapes=[pltpu.VMEM((2, f), jnp.float32)],
    )(h3, w, b, g, be)


def _bn_relu_body(ss_ref, y0_ref, y1_ref, o_ref):
    scale = ss_ref[0:1, :]
    shift = ss_ref[1:2, :]
    o_ref[:, 0, :] = jnp.maximum(y0_ref[...] * scale + shift, 0.0)
    o_ref[:, 1, :] = jnp.maximum(y1_ref[...] * scale + shift, 0.0)


def _tc_bn_relu_out(ss, y, nvalid, t_, br):
    f = y.shape[1]
    nb = nvalid // br
    return pl.pallas_call(
        _bn_relu_body,
        grid=(nb,),
        in_specs=[
            pl.BlockSpec((2, f), lambda i: (0, 0)),
            pl.BlockSpec((br, f), lambda i: (i, 0)),
            pl.BlockSpec((br, f), lambda i: (nb + i, 0)),
        ],
        out_specs=pl.BlockSpec((br, t_, f), lambda i: (i, 0, 0)),
        out_shape=jax.ShapeDtypeStruct((nvalid, t_, f), jnp.float32),
    )(ss, y, y)


def kernel(node_features, edge_index, edges_weight,
           W1, b1, g1, be1, W2, b2, g2, be2):
    n, t, f = node_features.shape
    e = edge_index.shape[1]
    npad = -(-n // (NS * ZROWS)) * (NS * ZROWS)
    nchunk = -(-(-(-e // (NS * CHUNK))) // 4) * 4
    epad = nchunk * NS * CHUNK
    br = 1000

    src = jnp.pad(edge_index[0], (0, epad - e))
    dst = jnp.pad(edge_index[1], (0, epad - e), constant_values=n)
    ew = jnp.pad(edges_weight, ((0, 0), (0, epad - e)))
    srcr1 = (jnp.stack([2 * src, 2 * src + 1])
             .reshape(t, NS, nchunk, CHUNK))
    srcr2 = jnp.stack([src, src + n]).reshape(t, NS, nchunk, CHUNK)
    dstr = dst.reshape(NS, nchunk, CHUNK)
    ewr = ew.reshape(t, NS, nchunk, CHUNK)

    x1 = node_features.reshape(n * t, f)
    ss0 = jnp.zeros((2, f), jnp.float32)

    h1 = _sc_scatter(x1, srcr1, dstr, ewr, ss0, npad, False)
    y1, ss1 = _tc_mm_stats(h1.reshape(t, npad, f), W1, b1.reshape(1, f),
                           g1.reshape(1, f), be1.reshape(1, f), n, br)

    h2 = _sc_scatter(y1, srcr2, dstr, ewr, ss1, npad, True)
    y2, ss2 = _tc_mm_stats(h2.reshape(t, npad, f), W2, b2.reshape(1, f),
                           g2.reshape(1, f), be2.reshape(1, f), n, br)

    return _tc_bn_relu_out(ss2, y2, n, t, br)


# speedup vs baseline: 1.4755x; 1.4755x over previous
import functools

import jax
import jax.numpy as jnp
from jax import lax
from jax.experimental import pallas as pl
from jax.experimental.pallas import tpu as pltpu
from jax.experimental.pallas import tpu_sc as plsc

EPS = 1e-5
NC = 2
NS = 16
LANES = 16
CHUNK = 128
ZROWS = 128


def _sc_scatter_body(npad, nchunk, nhalf, affine, nf,
                     x2, srcr, dstr, ewr, ss, h3,
                     acc, src_v, dst_v, ew_v, ss_v, rows0, rows1,
                     g0, g1, s0, s1):
    c = lax.axis_index("c")
    s = lax.axis_index("s")
    rows_per_tile = npad // NS
    nzcopy = rows_per_tile // ZROWS
    hchunk = nchunk // nhalf
    nvec = nf // LANES

    pltpu.sync_copy(ss, ss_v)

    zero16 = jnp.zeros((LANES,), jnp.float32)

    def zrow(i, carry):
        for k in range(nvec):
            rows0[i, pl.ds(k * LANES, LANES)] = zero16
        return carry

    lax.fori_loop(0, ZROWS, zrow, 0)
    base = s * rows_per_tile
    for z in range(nzcopy):
        pltpu.sync_copy(rows0, acc.at[pl.ds(base + z * ZROWS, ZROWS)])
    plsc.subcore_barrier()

    def scale_rows(buf, k):
        def group(g, gcarry):
            wv = ew_v[k, pl.ds(g * LANES, LANES)]
            if affine:
                for q in range(nvec):
                    sl = pl.ds(q * LANES, LANES)
                    sc = ss_v[0, sl]
                    sh = ss_v[1, sl]
                    for l in range(LANES):
                        i = g * LANES + l
                        v = jnp.maximum(buf[i, sl] * sc + sh, 0.0)
                        buf[i, sl] = v * wv[l]
            else:
                for l in range(LANES):
                    w = wv[l]
                    i = g * LANES + l
                    for q in range(nvec):
                        sl = pl.ds(q * LANES, LANES)
                        buf[i, sl] = buf[i, sl] * w
            return gcarry

        lax.fori_loop(0, CHUNK // LANES, group, 0)

    for half in range(nhalf):
        hbase = half * hchunk
        pltpu.sync_copy(srcr.at[c, s, pl.ds(hbase, hchunk)], src_v)
        pltpu.sync_copy(dstr.at[s, pl.ds(hbase, hchunk)], dst_v)
        pltpu.sync_copy(ewr.at[c, s, pl.ds(hbase, hchunk)], ew_v)

        pltpu.async_copy(x2.at[src_v.at[0]], rows0, g0)

        def pipe(k2, carry):
            k = 2 * k2
            @pl.when(k2 >= 1)
            def _():
                pltpu.make_async_copy(rows1, acc.at[dst_v.at[k - 1]],
                                      s1).wait()
            pltpu.async_copy(x2.at[src_v.at[k + 1]], rows1, g1)
            pltpu.make_async_copy(x2.at[src_v.at[k]], rows0, g0).wait()
            scale_rows(rows0, k)
            pltpu.async_copy(rows0, acc.at[dst_v.at[k]], s0, add=True)
            @pl.when(k2 < hchunk // 2 - 1)
            def _():
                pltpu.make_async_copy(rows0, acc.at[dst_v.at[k]], s0).wait()
                pltpu.async_copy(x2.at[src_v.at[k + 2]], rows0, g0)
            pltpu.make_async_copy(x2.at[src_v.at[k + 1]], rows1, g1).wait()
            scale_rows(rows1, k + 1)
            pltpu.async_copy(rows1, acc.at[dst_v.at[k + 1]], s1, add=True)
            return carry

        lax.fori_loop(0, hchunk // 2, pipe, 0)
        pltpu.make_async_copy(rows0, acc.at[dst_v.at[hchunk - 2]], s0).wait()
        pltpu.make_async_copy(rows1, acc.at[dst_v.at[hchunk - 1]], s1).wait()

    plsc.subcore_barrier()

    out_base = c * npad + base
    for z in range(nzcopy):
        pltpu.sync_copy(acc.at[pl.ds(base + z * ZROWS, ZROWS)],
                        h3.at[pl.ds(out_base + z * ZROWS, ZROWS)])


def _sc_scatter(x2, srcr, dstr, ewr, ss, npad, affine):
    f = x2.shape[1]
    nchunk = srcr.shape[2]
    nhalf = 2
    hchunk = nchunk // nhalf
    mesh = plsc.VectorSubcoreMesh(core_axis_name="c", subcore_axis_name="s",
                                  num_cores=NC, num_subcores=NS)
    return pl.kernel(
        functools.partial(_sc_scatter_body, npad, nchunk, nhalf, affine, f),
        out_type=jax.ShapeDtypeStruct((NC * npad, f), jnp.float32),
        mesh=mesh,
        scratch_types=[
            pltpu.VMEM_SHARED((npad, f), jnp.float32),
            pltpu.VMEM((hchunk, CHUNK), jnp.int32),
            pltpu.VMEM((hchunk, CHUNK), jnp.int32),
            pltpu.VMEM((hchunk, CHUNK), jnp.float32),
            pltpu.VMEM((2, f), jnp.float32),
            pltpu.VMEM((CHUNK, f), jnp.float32),
            pltpu.VMEM((CHUNK, f), jnp.float32),
            pltpu.SemaphoreType.DMA,
            pltpu.SemaphoreType.DMA,
            pltpu.SemaphoreType.DMA,
            pltpu.SemaphoreType.DMA,
        ],
    )(x2, srcr, dstr, ewr, ss)


def _mm_stats_body(cnt, x_ref, w_ref, b_ref, g_ref, be_ref,
                   y_ref, ss_ref, acc_ref):
    t = pl.program_id(0)
    i = pl.program_id(1)
    y = jnp.dot(x_ref[0], w_ref[...],
                preferred_element_type=jnp.float32) + b_ref[...]
    y_ref[...] = y

    @pl.when((t == 0) & (i == 0))
    def _():
        acc_ref[...] = jnp.zeros_like(acc_ref)

    acc_ref[0:1, :] += jnp.sum(y, axis=0, keepdims=True)
    acc_ref[1:2, :] += jnp.sum(y * y, axis=0, keepdims=True)

    @pl.when((t == pl.num_programs(0) - 1) & (i == pl.num_programs(1) - 1))
    def _():
        mean = acc_ref[0:1, :] / cnt
        var = acc_ref[1:2, :] / cnt - mean * mean
        scale = g_ref[...] * lax.rsqrt(var + EPS)
        ss_ref[0:1, :] = scale
        ss_ref[1:2, :] = be_ref[...] - mean * scale


def _tc_mm_stats(h3, w, b, g, be, nvalid, br):
    t_, npad, f = h3.shape
    grid = (t_, nvalid // br)
    return pl.pallas_call(
        functools.partial(_mm_stats_body, float(t_ * nvalid)),
        grid=grid,
        in_specs=[
            pl.BlockSpec((1, br, f), lambda t, i: (t, i, 0)),
            pl.BlockSpec((f, f), lambda t, i: (0, 0)),
            pl.BlockSpec((1, f), lambda t, i: (0, 0)),
            pl.BlockSpec((1, f), lambda t, i: (0, 0)),
            pl.BlockSpec((1, f), lambda t, i: (0, 0)),
        ],
        out_specs=[
            pl.BlockSpec((br, f), lambda t, i: (t * (nvalid // br) + i, 0)),
            pl.BlockSpec((2, f), lambda t, i: (0, 0)),
        ],
        out_shape=[
            jax.ShapeDtypeStruct((t_ * nvalid, f), jnp.float32),
            jax.ShapeDtypeStruct((2, f), jnp.float32),
        ],
        scratch_shapes=[pltpu.VMEM((2, f), jnp.float32)],
    )(h3, w, b, g, be)


def _bn_relu_body(ss_ref, y0_ref, y1_ref, o_ref):
    scale = ss_ref[0:1, :]
    shift = ss_ref[1:2, :]
    o_ref[:, 0, :] = jnp.maximum(y0_ref[...] * scale + shift, 0.0)
    o_ref[:, 1, :] = jnp.maximum(y1_ref[...] * scale + shift, 0.0)


def _tc_bn_relu_out(ss, y, nvalid, t_, br):
    f = y.shape[1]
    nb = nvalid // br
    return pl.pallas_call(
        _bn_relu_body,
        grid=(nb,),
        in_specs=[
            pl.BlockSpec((2, f), lambda i: (0, 0)),
            pl.BlockSpec((br, f), lambda i: (i, 0)),
            pl.BlockSpec((br, f), lambda i: (nb + i, 0)),
        ],
        out_specs=pl.BlockSpec((br, t_, f), lambda i: (i, 0, 0)),
        out_shape=jax.ShapeDtypeStruct((nvalid, t_, f), jnp.float32),
    )(ss, y, y)


def kernel(node_features, edge_index, edges_weight,
           W1, b1, g1, be1, W2, b2, g2, be2):
    n, t, f = node_features.shape
    e = edge_index.shape[1]
    npad = -(-n // (NS * ZROWS)) * (NS * ZROWS)
    nchunk = -(-(-(-e // (NS * CHUNK))) // 4) * 4
    epad = nchunk * NS * CHUNK
    br = 1000

    src = jnp.pad(edge_index[0], (0, epad - e))
    dst = jnp.pad(edge_index[1], (0, epad - e), constant_values=n)
    ew = jnp.pad(edges_weight, ((0, 0), (0, epad - e)))
    srcr1 = (jnp.stack([2 * src, 2 * src + 1])
             .reshape(t, NS, nchunk, CHUNK))
    srcr2 = jnp.stack([src, src + n]).reshape(t, NS, nchunk, CHUNK)
    dstr = dst.reshape(NS, nchunk, CHUNK)
    ewr = ew.reshape(t, NS, nchunk, CHUNK)

    x1 = node_features.reshape(n * t, f)
    ss0 = jnp.zeros((2, f), jnp.float32)

    h1 = _sc_scatter(x1, srcr1, dstr, ewr, ss0, npad, False)
    y1, ss1 = _tc_mm_stats(h1.reshape(t, npad, f), W1, b1.reshape(1, f),
                           g1.reshape(1, f), be1.reshape(1, f), n, br)

    h2 = _sc_scatter(y1, srcr2, dstr, ewr, ss1, npad, True)
    y2, ss2 = _tc_mm_stats(h2.reshape(t, npad, f), W2, b2.reshape(1, f),
                           g2.reshape(1, f), be2.reshape(1, f), n, br)

    return _tc_bn_relu_out(ss2, y2, n, t, br)
